# baseline (speedup 1.0000x reference)
import functools

import jax
import jax.numpy as jnp
from jax.experimental import pallas as pl
from jax.experimental.pallas import tpu as pltpu

N_TOK = 2048
D_IN = 768
HIDDEN = 24576
TOPK = 32


R_B1 = 256
H_B1 = 2048


def _encode_body(x_ref, w_ref, b_ref, p_ref):
    x = x_ref[...]
    w = w_ref[...]
    acc = jax.lax.dot_general(
        x, w, dimension_numbers=(((1,), (1,)), ((), ())),
        preferred_element_type=jnp.float32)
    acc = acc + b_ref[...]
    p_ref[...] = jnp.maximum(acc, 0.0)


def _encode(x, W_enc, b_enc):
    grid = (HIDDEN // H_B1, N_TOK // R_B1)
    return pl.pallas_call(
        _encode_body,
        grid=grid,
        in_specs=[
            pl.BlockSpec((R_B1, D_IN), lambda h, r: (r, 0)),
            pl.BlockSpec((H_B1, D_IN), lambda h, r: (h, 0)),
            pl.BlockSpec((1, H_B1), lambda h, r: (0, h)),
        ],
        out_specs=pl.BlockSpec((R_B1, H_B1), lambda h, r: (r, h)),
        out_shape=jax.ShapeDtypeStruct((N_TOK, HIDDEN), jnp.float32),
    )(x, W_enc, b_enc.reshape(1, HIDDEN))


R_B2 = 64
CW = 512
NCHUNK = HIDDEN // CW
NEG = float("-inf")
BIGI = 2**30


def _select_body(p_ref, out_ref, q_ref):
    def _init(c, _):
        sl = pl.ds(c * CW, CW)
        q_ref[:, sl] = p_ref[:, sl]
        out_ref[:, sl] = jnp.zeros((R_B2, CW), jnp.float32)
        return 0
    jax.lax.fori_loop(0, NCHUNK, _init, 0)

    iota = jax.lax.broadcasted_iota(jnp.int32, (R_B2, CW), 1)

    def _one_k(k, carry):
        prev_m, prev_g = carry

        def _chunk(c, inner):
            m, g = inner
            sl = pl.ds(c * CW, CW)
            blk = q_ref[:, sl]
            gpos = c * CW + iota
            hit = gpos == prev_g
            blk = jnp.where(hit, NEG, blk)
            q_ref[:, sl] = blk
            cm = jnp.max(blk, axis=1, keepdims=True)
            cidx = jnp.min(jnp.where(blk >= cm, gpos, BIGI), axis=1,
                           keepdims=True)
            upd = cm > m
            m = jnp.where(upd, cm, m)
            g = jnp.where(upd, cidx, g)
            return (m, g)

        m0 = jnp.full((R_B2, 1), NEG, jnp.float32)
        g0 = jnp.full((R_B2, 1), BIGI, jnp.int32)
        m, g = jax.lax.fori_loop(0, NCHUNK, _chunk, (m0, g0))

        def _write(c, _):
            sl = pl.ds(c * CW, CW)
            gpos = c * CW + iota
            sel = gpos == g
            out_ref[:, sl] = jnp.where(sel, m, out_ref[:, sl])
            return 0
        jax.lax.fori_loop(0, NCHUNK, _write, 0)
        return (m, g)

    m0 = jnp.full((R_B2, 1), NEG, jnp.float32)
    g0 = jnp.full((R_B2, 1), BIGI, jnp.int32)
    jax.lax.fori_loop(0, TOPK, _one_k, (m0, g0))


def _select(P):
    return pl.pallas_call(
        _select_body,
        grid=(N_TOK // R_B2,),
        in_specs=[pl.BlockSpec((R_B2, HIDDEN), lambda r: (r, 0))],
        out_specs=pl.BlockSpec((R_B2, HIDDEN), lambda r: (r, 0)),
        out_shape=jax.ShapeDtypeStruct((N_TOK, HIDDEN), jnp.float32),
        scratch_shapes=[pltpu.VMEM((R_B2, HIDDEN), jnp.float32)],
    )(P)


R_B3 = 256
H_B3 = 2048


def _decode_body(s_ref, w_ref, b_ref, o_ref):
    h = pl.program_id(1)

    @pl.when(h == 0)
    def _():
        o_ref[...] = jnp.zeros_like(o_ref)

    acc = jax.lax.dot_general(
        s_ref[...], w_ref[...], dimension_numbers=(((1,), (0,)), ((), ())),
        preferred_element_type=jnp.float32)
    o_ref[...] += acc

    @pl.when(h == (HIDDEN // H_B3) - 1)
    def _():
        o_ref[...] += b_ref[...]


def _decode(S, W_dec, b_dec):
    grid = (N_TOK // R_B3, HIDDEN // H_B3)
    return pl.pallas_call(
        _decode_body,
        grid=grid,
        in_specs=[
            pl.BlockSpec((R_B3, H_B3), lambda r, h: (r, h)),
            pl.BlockSpec((H_B3, D_IN), lambda r, h: (h, 0)),
            pl.BlockSpec((1, D_IN), lambda r, h: (0, 0)),
        ],
        out_specs=pl.BlockSpec((R_B3, D_IN), lambda r, h: (r, 0)),
        out_shape=jax.ShapeDtypeStruct((N_TOK, D_IN), jnp.float32),
    )(S, W_dec, b_dec.reshape(1, D_IN))


@jax.jit
def kernel(x, W_enc, b_enc, W_dec, b_dec):
    sae_in = x - b_dec
    P = _encode(sae_in, W_enc, b_enc)
    S = _select(P)
    return _decode(S, W_dec, b_dec)


# speedup vs baseline: 7.1711x; 7.1711x over previous
import functools

import jax
import jax.numpy as jnp
from jax import lax
from jax.experimental import pallas as pl
from jax.experimental.pallas import tpu as pltpu
from jax.experimental.pallas import tpu_sc as plsc

N_TOK = 2048
D_IN = 768
HIDDEN = 24576
TOPK = 32

LANES = 16
NWORK = 32
ROWS_PER_W = N_TOK // NWORK
NVEC = HIDDEN // LANES
CHUNK = 256
NCH = HIDDEN // CHUNK


R_B1 = 256
H_B1 = 2048


def _encode_body(x_ref, w_ref, b_ref, p_ref):
    acc = jax.lax.dot_general(
        x_ref[...], w_ref[...], dimension_numbers=(((1,), (1,)), ((), ())),
        preferred_element_type=jnp.float32)
    p_ref[...] = jnp.maximum(acc + b_ref[...], 0.0)


def _encode(x, W_enc, b_enc):
    grid = (HIDDEN // H_B1, N_TOK // R_B1)
    return pl.pallas_call(
        _encode_body,
        grid=grid,
        in_specs=[
            pl.BlockSpec((R_B1, D_IN), lambda h, r: (r, 0)),
            pl.BlockSpec((H_B1, D_IN), lambda h, r: (h, 0)),
            pl.BlockSpec((1, H_B1), lambda h, r: (0, h)),
        ],
        out_specs=pl.BlockSpec((R_B1, H_B1), lambda h, r: (r, h)),
        out_shape=jax.ShapeDtypeStruct((N_TOK, HIDDEN), jnp.float32),
    )(x, W_enc, b_enc.reshape(1, HIDDEN))


_GDN = jax.lax.GatherDimensionNumbers(
    offset_dims=(), collapsed_slice_dims=(0,), start_index_map=(0,))


def _splat(v, k):
    idx = jnp.full((LANES, 1), k, jnp.int32)
    return jax.lax.gather(v, idx, _GDN, (1,),
                          mode=jax.lax.GatherScatterMode.PROMISE_IN_BOUNDS)


def _shuf(v, idx):
    return jax.lax.gather(v, idx.reshape(LANES, 1), _GDN, (1,),
                          mode=jax.lax.GatherScatterMode.PROMISE_IN_BOUNDS)


def _allmax(v):
    lane = jax.lax.iota(jnp.int32, LANES)
    for s in (8, 4, 2, 1):
        v = jnp.maximum(v, _shuf(v, jnp.bitwise_xor(lane, s)))
    return v


def _allmin(v):
    lane = jax.lax.iota(jnp.int32, LANES)
    for s in (8, 4, 2, 1):
        v = jnp.minimum(v, _shuf(v, jnp.bitwise_xor(lane, s)))
    return v


def _scalar0(v):
    return jnp.squeeze(jax.lax.slice(v, (0,), (1,)))


def _sc_body(p_hbm, wdec_hbm, bdec_hbm, out_hbm,
             row_v, cm_v, sel_v, wrows_v, acc_v, bdec_v, wsplat_v, sem):
    wid = lax.axis_index("s") * 2 + lax.axis_index("c")
    lane = jnp.arange(LANES, dtype=jnp.int32)
    neg = jnp.float32(float("-inf"))

    pltpu.sync_copy(bdec_hbm, bdec_v)

    def one_row(i, _):
        t = wid * ROWS_PER_W + i
        pltpu.sync_copy(p_hbm.at[t], row_v)
        z = jnp.full((LANES,), neg, jnp.float32)

        def p1(c, _):
            m = row_v[pl.ds(c * CHUNK, LANES)]
            for u in range(1, CHUNK // LANES):
                m = jnp.maximum(m, row_v[pl.ds(c * CHUNK + u * LANES, LANES)])
            cm_v[pl.ds(c * LANES, LANES)] = m
            return 0
        lax.fori_loop(0, NCH, p1, 0)

        rv0 = z
        rv1 = z
        ri0 = jnp.zeros((LANES,), jnp.int32)
        ri1 = jnp.zeros((LANES,), jnp.int32)
        for k in range(TOPK):
            def scan(c, carry):
                bm, bc = carry
                v = cm_v[pl.ds(c * LANES, LANES)]
                gt = v > bm
                bm = jnp.where(gt, v, bm)
                bc = jnp.where(gt, c, bc)
                return bm, bc
            bm, bc = lax.fori_loop(0, NCH, scan,
                                   (z, jnp.zeros((LANES,), jnp.int32)))
            ms = _allmax(bm)
            cstar = _scalar0(_allmin(jnp.where(bm == ms, bc, 2**30)))
            base = cstar * CHUNK

            def mp1(u, p):
                v = row_v[pl.ds(base + u * LANES, LANES)]
                eq = v == ms
                return jnp.minimum(p, jnp.where(eq, u * LANES + lane, 2**30))
            p = lax.fori_loop(0, CHUNK // LANES, mp1,
                              jnp.full((LANES,), 2**30, jnp.int32))
            pos = _allmin(p)

            def mp2(u, nm):
                sl = pl.ds(base + u * LANES, LANES)
                v = row_v[sl]
                hit = (u * LANES + lane) == pos
                v = jnp.where(hit, neg, v)
                row_v[sl] = v
                return jnp.maximum(nm, v)
            nm = lax.fori_loop(0, CHUNK // LANES, mp2, z)
            cm_v[pl.ds(cstar * LANES, LANES)] = nm

            mi = base + pos
            if k < 16:
                rv0 = jnp.where(lane == k, ms, rv0)
                ri0 = jnp.where(lane == k, mi, ri0)
            else:
                rv1 = jnp.where(lane == k - 16, ms, rv1)
                ri1 = jnp.where(lane == k - 16, mi, ri1)

        sel_v[pl.ds(0, LANES)] = ri0
        sel_v[pl.ds(LANES, LANES)] = ri1
        pltpu.async_copy(wdec_hbm.at[sel_v], wrows_v, sem).wait()

        for k in range(TOPK):
            w = _splat(rv0, k) if k < 16 else _splat(rv1, k - 16)
            wsplat_v[pl.ds(k * LANES, LANES)] = w

        def dec_j(j, _):
            sl = pl.ds(j * LANES, LANES)
            a = bdec_v[sl]
            for k in range(TOPK):
                a = a + wsplat_v[pl.ds(k * LANES, LANES)] * wrows_v[k, sl]
            acc_v[sl] = a
            return 0
        lax.fori_loop(0, D_IN // LANES, dec_j, 0)
        pltpu.sync_copy(acc_v, out_hbm.at[t])
        return 0

    lax.fori_loop(0, ROWS_PER_W, one_row, 0)


def _sc_topk_decode(P, W_dec, b_dec):
    mesh = plsc.VectorSubcoreMesh(core_axis_name="c", subcore_axis_name="s")
    fn = pl.kernel(
        _sc_body, mesh=mesh,
        out_type=jax.ShapeDtypeStruct((N_TOK, D_IN), jnp.float32),
        scratch_types=[
            pltpu.VMEM((HIDDEN,), jnp.float32),
            pltpu.VMEM((NCH * LANES,), jnp.float32),
            pltpu.VMEM((TOPK,), jnp.int32),
            pltpu.VMEM((TOPK, D_IN), jnp.float32),
            pltpu.VMEM((D_IN,), jnp.float32),
            pltpu.VMEM((D_IN,), jnp.float32),
            pltpu.VMEM((TOPK * LANES,), jnp.float32),
            pltpu.SemaphoreType.DMA,
        ],
    )
    return fn(P, W_dec, b_dec)


@jax.jit
def kernel(x, W_enc, b_enc, W_dec, b_dec):
    sae_in = x - b_dec
    P = _encode(sae_in, W_enc, b_enc)
    return _sc_topk_decode(P, W_dec, b_dec)


# speedup vs baseline: 17.1965x; 2.3980x over previous
import functools

import jax
import jax.numpy as jnp
from jax import lax
from jax.experimental import pallas as pl
from jax.experimental.pallas import tpu as pltpu
from jax.experimental.pallas import tpu_sc as plsc

N_TOK = 2048
D_IN = 768
HIDDEN = 24576
TOPK = 32

LANES = 16
NWORK = 32
ROWS_PER_W = N_TOK // NWORK
NVEC = HIDDEN // LANES
CHUNK = 256
NCH = HIDDEN // CHUNK
NSUP = NCH // 16


R_B1 = 256
H_B1 = 2048


def _encode_body(x_ref, w_ref, b_ref, p_ref):
    acc = jax.lax.dot_general(
        x_ref[...], w_ref[...], dimension_numbers=(((1,), (1,)), ((), ())),
        preferred_element_type=jnp.float32)
    p_ref[...] = jnp.maximum(acc + b_ref[...], 0.0)


def _encode(x, W_enc, b_enc):
    grid = (HIDDEN // H_B1, N_TOK // R_B1)
    return pl.pallas_call(
        _encode_body,
        grid=grid,
        in_specs=[
            pl.BlockSpec((R_B1, D_IN), lambda h, r: (r, 0)),
            pl.BlockSpec((H_B1, D_IN), lambda h, r: (h, 0)),
            pl.BlockSpec((1, H_B1), lambda h, r: (0, h)),
        ],
        out_specs=pl.BlockSpec((R_B1, H_B1), lambda h, r: (r, h)),
        out_shape=jax.ShapeDtypeStruct((N_TOK, HIDDEN), jnp.float32),
    )(x, W_enc, b_enc.reshape(1, HIDDEN))


_GDN = jax.lax.GatherDimensionNumbers(
    offset_dims=(), collapsed_slice_dims=(0,), start_index_map=(0,))


def _splat(v, k):
    idx = jnp.full((LANES, 1), k, jnp.int32)
    return jax.lax.gather(v, idx, _GDN, (1,),
                          mode=jax.lax.GatherScatterMode.PROMISE_IN_BOUNDS)


def _shuf(v, idx):
    return jax.lax.gather(v, idx.reshape(LANES, 1), _GDN, (1,),
                          mode=jax.lax.GatherScatterMode.PROMISE_IN_BOUNDS)


def _allmax(v):
    lane = jax.lax.iota(jnp.int32, LANES)
    for s in (8, 4, 2, 1):
        v = jnp.maximum(v, _shuf(v, jnp.bitwise_xor(lane, s)))
    return v


def _allmin(v):
    lane = jax.lax.iota(jnp.int32, LANES)
    for s in (8, 4, 2, 1):
        v = jnp.minimum(v, _shuf(v, jnp.bitwise_xor(lane, s)))
    return v


def _scalar0(v):
    return jnp.squeeze(jax.lax.slice(v, (0,), (1,)))


def _sc_body(p_hbm, wdec_hbm, bdec_hbm, out_hbm,
             rowa_v, rowb_v, cm_v, scm_v, sel_v, wrows_v, acc_v, bdec_v,
             wsplat_v, sema, semb, wsem):
    wid = lax.axis_index("s") * 2 + lax.axis_index("c")
    lane = jnp.arange(LANES, dtype=jnp.int32)
    neg = jnp.float32(float("-inf"))

    pltpu.sync_copy(bdec_hbm, bdec_v)

    z = jnp.full((LANES,), neg, jnp.float32)
    zi = jnp.zeros((LANES,), jnp.int32)
    big = jnp.full((LANES,), 2**30, jnp.int32)

    def process(row_v, t):
        def p1(c2, _):
            for q in range(2):
                c = c2 * 2 + q
                m = row_v[pl.ds(c * CHUNK, LANES)]
                for u in range(1, CHUNK // LANES):
                    m = jnp.maximum(
                        m, row_v[pl.ds(c * CHUNK + u * LANES, LANES)])
                cm_v[pl.ds(c * LANES, LANES)] = m
            return 0
        lax.fori_loop(0, NCH // 2, p1, 0)

        for s in range(NSUP):
            m = cm_v[pl.ds(s * 16 * LANES, LANES)]
            for u in range(1, 16):
                m = jnp.maximum(m, cm_v[pl.ds((s * 16 + u) * LANES, LANES)])
            scm_v[pl.ds(s * LANES, LANES)] = m

        def one_k(k, carry):
            rv0, rv1, ri0, ri1 = carry
            bm = scm_v[pl.ds(0, LANES)]
            bs = zi
            for s in range(1, NSUP):
                v = scm_v[pl.ds(s * LANES, LANES)]
                gt = v > bm
                bm = jnp.where(gt, v, bm)
                bs = jnp.where(gt, s, bs)
            ms = _allmax(bm)
            sstar = _scalar0(_allmin(jnp.where(bm == ms, bs, big)))
            gbase = sstar * 16 * LANES
            bm2 = cm_v[pl.ds(gbase, LANES)]
            bc2 = zi
            for u in range(1, 16):
                v = cm_v[pl.ds(gbase + u * LANES, LANES)]
                gt = v > bm2
                bm2 = jnp.where(gt, v, bm2)
                bc2 = jnp.where(gt, u, bc2)
            cstar = sstar * 16 + _scalar0(
                _allmin(jnp.where(bm2 == ms, bc2, big)))
            base = cstar * CHUNK

            p = big
            for u in range(CHUNK // LANES):
                v = row_v[pl.ds(base + u * LANES, LANES)]
                p = jnp.minimum(p, jnp.where(v == ms, u * LANES + lane, big))
            pos = _allmin(p)

            nm = z
            for u in range(CHUNK // LANES):
                sl = pl.ds(base + u * LANES, LANES)
                v = row_v[sl]
                v = jnp.where((u * LANES + lane) == pos, neg, v)
                row_v[sl] = v
                nm = jnp.maximum(nm, v)
            cm_v[pl.ds(cstar * LANES, LANES)] = nm
            sm = cm_v[pl.ds(gbase, LANES)]
            for u in range(1, 16):
                sm = jnp.maximum(sm, cm_v[pl.ds(gbase + u * LANES, LANES)])
            scm_v[pl.ds(sstar * LANES, LANES)] = sm

            mi = base + pos
            wsplat_v[pl.ds(k * LANES, LANES)] = ms
            rv0 = jnp.where(lane == k, ms, rv0)
            ri0 = jnp.where(lane == k, mi, ri0)
            rv1 = jnp.where(lane == k - 16, ms, rv1)
            ri1 = jnp.where(lane == k - 16, mi, ri1)
            return rv0, rv1, ri0, ri1

        rv0, rv1, ri0, ri1 = lax.fori_loop(0, TOPK, one_k, (z, z, zi, zi))
        return rv0, rv1, ri0, ri1

    def decode(t, ri0, ri1):
        sel_v[pl.ds(0, LANES)] = ri0
        sel_v[pl.ds(LANES, LANES)] = ri1
        pltpu.async_copy(wdec_hbm.at[sel_v], wrows_v, wsem).wait()

        def dec_j(j, _):
            sl = pl.ds(j * LANES, LANES)
            a = bdec_v[sl]
            for k in range(TOPK):
                a = a + wsplat_v[pl.ds(k * LANES, LANES)] * wrows_v[k, sl]
            acc_v[sl] = a
            return 0
        lax.fori_loop(0, D_IN // LANES, dec_j, 0)
        pltpu.sync_copy(acc_v, out_hbm.at[t])

    t0 = wid * ROWS_PER_W
    pltpu.async_copy(p_hbm.at[t0], rowa_v, sema)

    def two_rows(ii, _):
        ta = t0 + 2 * ii
        pltpu.make_async_copy(p_hbm.at[ta], rowa_v, sema).wait()
        pltpu.async_copy(p_hbm.at[ta + 1], rowb_v, semb)
        _, _, ri0, ri1 = process(rowa_v, ta)
        decode(ta, ri0, ri1)
        pltpu.make_async_copy(p_hbm.at[ta + 1], rowb_v, semb).wait()

        @pl.when(ii < ROWS_PER_W // 2 - 1)
        def _():
            pltpu.async_copy(p_hbm.at[ta + 2], rowa_v, sema)
        _, _, ri0b, ri1b = process(rowb_v, ta + 1)
        decode(ta + 1, ri0b, ri1b)
        return 0

    lax.fori_loop(0, ROWS_PER_W // 2, two_rows, 0)


def _sc_topk_decode(P, W_dec, b_dec):
    mesh = plsc.VectorSubcoreMesh(core_axis_name="c", subcore_axis_name="s")
    fn = pl.kernel(
        _sc_body, mesh=mesh,
        out_type=jax.ShapeDtypeStruct((N_TOK, D_IN), jnp.float32),
        scratch_types=[
            pltpu.VMEM((HIDDEN,), jnp.float32),
            pltpu.VMEM((HIDDEN,), jnp.float32),
            pltpu.VMEM((NCH * LANES,), jnp.float32),
            pltpu.VMEM((NSUP * LANES,), jnp.float32),
            pltpu.VMEM((TOPK,), jnp.int32),
            pltpu.VMEM((TOPK, D_IN), jnp.float32),
            pltpu.VMEM((D_IN,), jnp.float32),
            pltpu.VMEM((D_IN,), jnp.float32),
            pltpu.VMEM((TOPK * LANES,), jnp.float32),
            pltpu.SemaphoreType.DMA,
            pltpu.SemaphoreType.DMA,
            pltpu.SemaphoreType.DMA,
        ],
    )
    return fn(P, W_dec, b_dec)


@jax.jit
def kernel(x, W_enc, b_enc, W_dec, b_dec):
    sae_in = x - b_dec
    P = _encode(sae_in, W_enc, b_enc)
    return _sc_topk_decode(P, W_dec, b_dec)
